# Initial kernel scaffold; baseline (speedup 1.0000x reference)
import jax
import jax.numpy as jnp
from jax.experimental import pallas as pl


def kernel(feature, edge_index, W1, a1, W2, a2, W3, a3, W4, a4):
    raise NotImplementedError("write your pallas kernel here")


# speedup vs baseline: 16.9588x; 16.9588x over previous
import functools

import jax
import jax.numpy as jnp
from jax import lax
from jax.experimental import pallas as pl
from jax.experimental.pallas import tpu as pltpu
from jax.experimental.pallas import tpu_sc as plsc

N = 10000
E = 320000
D = 128
TW = 144
NW = 32
CW = 128
NCH = 79
EPT = NCH * CW
EPAD = NW * EPT
NA = 10240
RPS = NA // 16
ZB = 128


def _proj_body(h, w_ref, a_ref, zt_ref, ed_ref):
    z = jnp.dot(h, w_ref[...], preferred_element_type=jnp.float32)
    e2 = jnp.dot(z, a_ref[...], preferred_element_type=jnp.float32)
    zt_ref[:, :D] = z
    zt_ref[:, D:] = jnp.broadcast_to(e2[:, 0:1], (z.shape[0], TW - D))
    ed_ref[...] = e2[:, 1:2]


def _tc_proj_kernel(h_ref, w_ref, a_ref, zt_ref, ed_ref):
    _proj_body(h_ref[...], w_ref, a_ref, zt_ref, ed_ref)


def _combine(p_ref):
    num = p_ref[0, :, :D] + p_ref[1, :, :D]
    den = p_ref[0, :, D] + p_ref[1, :, D]
    return num / (den + 1e-9)[:, None]


def _tc_comb_proj_kernel(p_ref, w_ref, a_ref, zt_ref, ed_ref):
    h = jnp.maximum(_combine(p_ref), 0.0)
    _proj_body(h, w_ref, a_ref, zt_ref, ed_ref)


def _tc_comb_last_kernel(p_ref, h_ref):
    h_ref[...] = _combine(p_ref)


_B = 1000


def _tc_proj(h, w, a2):
    return pl.pallas_call(
        _tc_proj_kernel,
        grid=(N // _B,),
        in_specs=[
            pl.BlockSpec((_B, D), lambda i: (i, 0)),
            pl.BlockSpec((D, D), lambda i: (0, 0)),
            pl.BlockSpec((D, 2), lambda i: (0, 0)),
        ],
        out_specs=[
            pl.BlockSpec((_B, TW), lambda i: (i, 0)),
            pl.BlockSpec((_B, 1), lambda i: (i, 0)),
        ],
        out_shape=[
            jax.ShapeDtypeStruct((N, TW), jnp.float32),
            jax.ShapeDtypeStruct((N, 1), jnp.float32),
        ],
    )(h, w, a2)


def _tc_comb_proj(p, w, a2):
    return pl.pallas_call(
        _tc_comb_proj_kernel,
        grid=(N // _B,),
        in_specs=[
            pl.BlockSpec((2, _B, TW), lambda i: (0, i, 0)),
            pl.BlockSpec((D, D), lambda i: (0, 0)),
            pl.BlockSpec((D, 2), lambda i: (0, 0)),
        ],
        out_specs=[
            pl.BlockSpec((_B, TW), lambda i: (i, 0)),
            pl.BlockSpec((_B, 1), lambda i: (i, 0)),
        ],
        out_shape=[
            jax.ShapeDtypeStruct((N, TW), jnp.float32),
            jax.ShapeDtypeStruct((N, 1), jnp.float32),
        ],
    )(p, w, a2)


def _tc_comb_last(p):
    return pl.pallas_call(
        _tc_comb_last_kernel,
        grid=(N // _B,),
        in_specs=[pl.BlockSpec((2, _B, TW), lambda i: (0, i, 0))],
        out_specs=pl.BlockSpec((_B, D), lambda i: (i, 0)),
        out_shape=jax.ShapeDtypeStruct((N, D), jnp.float32),
    )(p)


@functools.partial(
    pl.kernel,
    out_type=jax.ShapeDtypeStruct((2, NA, TW), jnp.float32),
    mesh=plsc.VectorSubcoreMesh(core_axis_name="c", subcore_axis_name="s"),
    compiler_params=pltpu.CompilerParams(
        use_tc_tiling_on_sc=False, needs_layout_passes=False),
    scratch_types=[
        pltpu.VMEM((CW,), jnp.int32),
        pltpu.VMEM((CW,), jnp.int32),
        pltpu.VMEM((CW, TW), jnp.float32),
        pltpu.VMEM((CW,), jnp.float32),
        pltpu.VMEM((N,), jnp.float32),
        pltpu.VMEM_SHARED((NA, TW), jnp.float32),
    ],
)
def _sc_aggregate(ztab, edt, src2d, dst2d, out, srcj, dstj, rows, exv,
                  edtab, acc):
    c = lax.axis_index("c")
    s = lax.axis_index("s")
    wid = s * 2 + c
    iota16 = lax.iota(jnp.int32, 16)
    zero16 = jnp.zeros((16,), jnp.float32)

    def _zrow(i, carry):
        for r in range(TW // 16):
            rows[i, pl.ds(r * 16, 16)] = zero16
        return carry

    lax.fori_loop(0, CW, _zrow, 0)
    for t in range(RPS // ZB):
        pltpu.sync_copy(rows, acc.at[pl.ds(s * RPS + t * ZB, ZB)])
    plsc.subcore_barrier()

    pltpu.sync_copy(edt, edtab)

    def _chunk(j, carry):
        pltpu.sync_copy(src2d.at[wid, j], srcj)
        pltpu.sync_copy(dst2d.at[wid, j], dstj)
        pltpu.sync_copy(ztab.at[srcj], rows)
        gbase = wid * EPT + j * CW
        for k in range(CW // 16):
            es16 = plsc.load_gather(
                rows, [iota16 + (k * 16), jnp.full((16,), D, jnp.int32)])
            d16 = plsc.load_gather(edtab, [dstj[pl.ds(k * 16, 16)]])
            raw = es16 + d16
            e16 = jnp.where(raw > 0, raw, 0.2 * raw)
            live = (gbase + k * 16 + iota16) < E
            exv[pl.ds(k * 16, 16)] = jnp.where(live, jnp.exp(e16), 0.0)

        def _scale(ei, carry2):
            bc = plsc.load_gather(exv, [jnp.full((16,), 0, jnp.int32) + ei])
            for r in range(D // 16):
                rows[ei, pl.ds(r * 16, 16)] = rows[ei, pl.ds(r * 16, 16)] * bc
            rows[ei, pl.ds(D, 16)] = bc
            return carry2

        lax.fori_loop(0, CW, _scale, 0)
        pltpu.sync_copy(rows, acc.at[dstj], add=True)
        return carry

    lax.fori_loop(0, NCH, _chunk, 0)
    plsc.subcore_barrier()

    for t in range(RPS // ZB):
        rs = s * RPS + t * ZB
        pltpu.sync_copy(acc.at[pl.ds(rs, ZB)], rows)
        pltpu.sync_copy(rows, out.at[c, pl.ds(rs, ZB)])


def kernel(feature, edge_index, W1, a1, W2, a2, W3, a3, W4, a4):
    pad = EPAD - E
    src2d = jnp.pad(edge_index[0], (0, pad)).reshape(NW, NCH, CW)
    dst2d = jnp.pad(edge_index[1], (0, pad)).reshape(NW, NCH, CW)

    def a2col(a):
        return jnp.stack([a[:D], a[D:]], axis=1)

    zt, ed = _tc_proj(feature, W1, a2col(a1))
    p = _sc_aggregate(zt, ed[:, 0], src2d, dst2d)
    for w, a in ((W2, a2), (W3, a3), (W4, a4)):
        zt, ed = _tc_comb_proj(p, w, a2col(a))
        p = _sc_aggregate(zt, ed[:, 0], src2d, dst2d)
    return _tc_comb_last(p)


# speedup vs baseline: 25.4681x; 1.5018x over previous
import functools

import jax
import jax.numpy as jnp
from jax import lax
from jax.experimental import pallas as pl
from jax.experimental.pallas import tpu as pltpu
from jax.experimental.pallas import tpu_sc as plsc

N = 10000
E = 320000
D = 128
TW = 144
NW = 32
CW = 128
NCH = 79
EPT = NCH * CW
EPAD = NW * EPT
NA = 10112
RPS = NA // 16
HW = CW // 2


def _proj_body(h, w_ref, a_ref, zt_ref, ed_ref):
    z = jnp.dot(h, w_ref[...], preferred_element_type=jnp.float32)
    e2 = jnp.dot(z, a_ref[...], preferred_element_type=jnp.float32)
    zt_ref[:, :D] = z
    zt_ref[:, D:] = jnp.broadcast_to(e2[:, 0:1], (z.shape[0], TW - D))
    ed_ref[...] = e2[:, 1:2]


def _tc_proj_kernel(h_ref, w_ref, a_ref, zt_ref, ed_ref):
    _proj_body(h_ref[...], w_ref, a_ref, zt_ref, ed_ref)


def _combine(p_ref):
    num = p_ref[0, :, :D] + p_ref[1, :, :D]
    den = p_ref[0, :, D] + p_ref[1, :, D]
    return num / (den + 1e-9)[:, None]


def _tc_comb_proj_kernel(p_ref, w_ref, a_ref, zt_ref, ed_ref):
    h = jnp.maximum(_combine(p_ref), 0.0)
    _proj_body(h, w_ref, a_ref, zt_ref, ed_ref)


def _tc_comb_last_kernel(p_ref, h_ref):
    h_ref[...] = _combine(p_ref)


_B = 1000


def _tc_proj(h, w, a2):
    return pl.pallas_call(
        _tc_proj_kernel,
        grid=(N // _B,),
        in_specs=[
            pl.BlockSpec((_B, D), lambda i: (i, 0)),
            pl.BlockSpec((D, D), lambda i: (0, 0)),
            pl.BlockSpec((D, 2), lambda i: (0, 0)),
        ],
        out_specs=[
            pl.BlockSpec((_B, TW), lambda i: (i, 0)),
            pl.BlockSpec((_B, 1), lambda i: (i, 0)),
        ],
        out_shape=[
            jax.ShapeDtypeStruct((N, TW), jnp.float32),
            jax.ShapeDtypeStruct((N, 1), jnp.float32),
        ],
    )(h, w, a2)


def _tc_comb_proj(p, w, a2):
    return pl.pallas_call(
        _tc_comb_proj_kernel,
        grid=(N // _B,),
        in_specs=[
            pl.BlockSpec((2, _B, TW), lambda i: (0, i, 0)),
            pl.BlockSpec((D, D), lambda i: (0, 0)),
            pl.BlockSpec((D, 2), lambda i: (0, 0)),
        ],
        out_specs=[
            pl.BlockSpec((_B, TW), lambda i: (i, 0)),
            pl.BlockSpec((_B, 1), lambda i: (i, 0)),
        ],
        out_shape=[
            jax.ShapeDtypeStruct((N, TW), jnp.float32),
            jax.ShapeDtypeStruct((N, 1), jnp.float32),
        ],
    )(p, w, a2)


def _tc_comb_last(p):
    return pl.pallas_call(
        _tc_comb_last_kernel,
        grid=(N // _B,),
        in_specs=[pl.BlockSpec((2, _B, TW), lambda i: (0, i, 0))],
        out_specs=pl.BlockSpec((_B, D), lambda i: (i, 0)),
        out_shape=jax.ShapeDtypeStruct((N, D), jnp.float32),
    )(p)


@functools.partial(
    pl.kernel,
    out_type=jax.ShapeDtypeStruct((2, NA, TW), jnp.float32),
    mesh=plsc.VectorSubcoreMesh(core_axis_name="c", subcore_axis_name="s"),
    compiler_params=pltpu.CompilerParams(
        use_tc_tiling_on_sc=False, needs_layout_passes=False),
    scratch_types=[
        pltpu.VMEM((NCH, CW), jnp.int32),
        pltpu.VMEM((NCH, CW), jnp.int32),
        pltpu.VMEM((CW, TW), jnp.float32),
        pltpu.VMEM((CW,), jnp.float32),
        pltpu.VMEM((2, HW), jnp.float32),
        pltpu.VMEM_SHARED((NA, TW), jnp.float32),
        pltpu.SemaphoreType.DMA,
        pltpu.SemaphoreType.DMA,
        pltpu.SemaphoreType.DMA,
        pltpu.SemaphoreType.DMA,
    ],
)
def _sc_aggregate(ztab, edt, src2d, dst2d, out, srcv, dstv, rows, exv,
                  edh, acc, semg0, semg1, sems0, sems1):
    c = lax.axis_index("c")
    s = lax.axis_index("s")
    wid = s * 2 + c
    iota16 = lax.iota(jnp.int32, 16)
    zero16 = jnp.zeros((16,), jnp.float32)
    semg = (semg0, semg1)
    sems = (sems0, sems1)

    def _zrow(i, carry):
        for r in range(TW // 16):
            rows[i, pl.ds(r * 16, 16)] = zero16
        return carry

    lax.fori_loop(0, CW, _zrow, 0)
    pieces = [(t * CW, CW) for t in range(RPS // CW)] + [
        (RPS // CW * CW, RPS % CW)]
    for rs, rn in pieces:
        pltpu.sync_copy(rows.at[pl.ds(0, rn)],
                        acc.at[pl.ds(s * RPS + rs, rn)])
    plsc.subcore_barrier()

    pltpu.sync_copy(src2d.at[wid], srcv)
    pltpu.sync_copy(dst2d.at[wid], dstv)

    def _gather_half(j, b, buf):
        pltpu.async_copy(ztab.at[srcv.at[j, pl.ds(b * HW, HW)]],
                         rows.at[pl.ds(buf * HW, HW)], semg[buf])
        pltpu.async_copy(edt.at[dstv.at[j, pl.ds(b * HW, HW)]],
                         edh.at[buf], semg[buf])

    def _wait_gather(buf):
        pltpu.make_async_copy(ztab.at[srcv.at[0, pl.ds(0, HW)]],
                              rows.at[pl.ds(buf * HW, HW)],
                              semg[buf]).wait()
        pltpu.make_async_copy(edt.at[dstv.at[0, pl.ds(0, HW)]],
                              edh.at[buf], semg[buf]).wait()

    def _start_scatter(j, b, buf):
        pltpu.async_copy(rows.at[pl.ds(buf * HW, HW)],
                         acc.at[dstv.at[j, pl.ds(b * HW, HW)]],
                         sems[buf], add=True)

    def _wait_scatter(buf):
        pltpu.make_async_copy(rows.at[pl.ds(buf * HW, HW)],
                              acc.at[dstv.at[0, pl.ds(0, HW)]],
                              sems[buf]).wait()

    _gather_half(0, 0, 0)

    def _chunk(j, carry):
        for b in range(2):
            if b == 0:
                @pl.when(j > 0)
                def _():
                    _wait_scatter(1)
                _gather_half(j, 1, 1)
            else:
                _wait_scatter(0)
                nj = jnp.minimum(j + 1, NCH - 1)

                @pl.when(j < NCH - 1)
                def _():
                    _gather_half(nj, 0, 0)
            _wait_gather(b)

            gbase = wid * EPT + j * CW + b * HW
            for k in range(HW // 16):
                es16 = plsc.load_gather(
                    rows, [iota16 + (b * HW + k * 16),
                           jnp.full((16,), D, jnp.int32)])
                d16 = edh[b, pl.ds(k * 16, 16)]
                raw = es16 + d16
                e16 = jnp.where(raw > 0, raw, 0.2 * raw)
                live = (gbase + k * 16 + iota16) < E
                exv[pl.ds(b * HW + k * 16, 16)] = jnp.where(
                    live, jnp.exp(e16), 0.0)

            def _scale(ei, carry2):
                bc = plsc.load_gather(
                    exv, [jnp.full((16,), 0, jnp.int32) + ei])
                for r in range(D // 16):
                    rows[ei, pl.ds(r * 16, 16)] = (
                        rows[ei, pl.ds(r * 16, 16)] * bc)
                rows[ei, pl.ds(D, 16)] = bc
                return carry2

            lax.fori_loop(b * HW, b * HW + HW, _scale, 0)
            _start_scatter(j, b, b)
        return carry

    lax.fori_loop(0, NCH, _chunk, 0)
    _wait_scatter(1)
    plsc.subcore_barrier()

    for rs, rn in pieces:
        pltpu.sync_copy(acc.at[pl.ds(s * RPS + rs, rn)],
                        rows.at[pl.ds(0, rn)])
        pltpu.sync_copy(rows.at[pl.ds(0, rn)],
                        out.at[c, pl.ds(s * RPS + rs, rn)])


def kernel(feature, edge_index, W1, a1, W2, a2, W3, a3, W4, a4):
    pad = EPAD - E
    src2d = jnp.pad(edge_index[0], (0, pad)).reshape(NW, NCH, CW)
    dst2d = jnp.pad(edge_index[1], (0, pad)).reshape(NW, NCH, CW)

    def a2col(a):
        return jnp.stack([a[:D], a[D:]], axis=1)

    zt, ed = _tc_proj(feature, W1, a2col(a1))
    p = _sc_aggregate(zt, ed[:, 0], src2d, dst2d)
    for w, a in ((W2, a2), (W3, a3), (W4, a4)):
        zt, ed = _tc_comb_proj(p, w, a2col(a))
        p = _sc_aggregate(zt, ed[:, 0], src2d, dst2d)
    return _tc_comb_last(p)


# speedup vs baseline: 29.0338x; 1.1400x over previous
import functools

import jax
import jax.numpy as jnp
from jax import lax
from jax.experimental import pallas as pl
from jax.experimental.pallas import tpu as pltpu
from jax.experimental.pallas import tpu_sc as plsc

N = 10000
E = 320000
D = 128
TW = 144
CW = 128
HW = CW // 2
NCH0 = 96
NCH1 = 62
NCHX = NCH0
SPLIT = 16 * NCH0 * CW
EPAD = 16 * (NCH0 + NCH1) * CW
NA = 10112
RPS = NA // 16


def _proj_body(h, w_ref, a_ref, zt_ref, ed_ref):
    z = jnp.dot(h, w_ref[...], preferred_element_type=jnp.float32)
    e2 = jnp.dot(z, a_ref[...], preferred_element_type=jnp.float32)
    zt_ref[:, :D] = z
    zt_ref[:, D:] = jnp.broadcast_to(e2[:, 0:1], (z.shape[0], TW - D))
    ed_ref[...] = e2[:, 1:2]


def _tc_proj_kernel(h_ref, w_ref, a_ref, zt_ref, ed_ref):
    _proj_body(h_ref[...], w_ref, a_ref, zt_ref, ed_ref)


def _combine(p_ref):
    num = p_ref[0, :, :D] + p_ref[1, :, :D]
    den = p_ref[0, :, D] + p_ref[1, :, D]
    return num / (den + 1e-9)[:, None]


def _tc_comb_proj_kernel(p_ref, w_ref, a_ref, zt_ref, ed_ref):
    h = jnp.maximum(_combine(p_ref), 0.0)
    _proj_body(h, w_ref, a_ref, zt_ref, ed_ref)


def _tc_comb_last_kernel(p_ref, h_ref):
    h_ref[...] = _combine(p_ref)


_B = 1000


def _tc_proj(h, w, a2):
    return pl.pallas_call(
        _tc_proj_kernel,
        grid=(N // _B,),
        in_specs=[
            pl.BlockSpec((_B, D), lambda i: (i, 0)),
            pl.BlockSpec((D, D), lambda i: (0, 0)),
            pl.BlockSpec((D, 2), lambda i: (0, 0)),
        ],
        out_specs=[
            pl.BlockSpec((_B, TW), lambda i: (i, 0)),
            pl.BlockSpec((_B, 1), lambda i: (i, 0)),
        ],
        out_shape=[
            jax.ShapeDtypeStruct((N, TW), jnp.float32),
            jax.ShapeDtypeStruct((N, 1), jnp.float32),
        ],
    )(h, w, a2)


def _tc_comb_proj(p, w, a2):
    return pl.pallas_call(
        _tc_comb_proj_kernel,
        grid=(N // _B,),
        in_specs=[
            pl.BlockSpec((2, _B, TW), lambda i: (0, i, 0)),
            pl.BlockSpec((D, D), lambda i: (0, 0)),
            pl.BlockSpec((D, 2), lambda i: (0, 0)),
        ],
        out_specs=[
            pl.BlockSpec((_B, TW), lambda i: (i, 0)),
            pl.BlockSpec((_B, 1), lambda i: (i, 0)),
        ],
        out_shape=[
            jax.ShapeDtypeStruct((N, TW), jnp.float32),
            jax.ShapeDtypeStruct((N, 1), jnp.float32),
        ],
    )(p, w, a2)


def _tc_comb_last(p):
    return pl.pallas_call(
        _tc_comb_last_kernel,
        grid=(N // _B,),
        in_specs=[pl.BlockSpec((2, _B, TW), lambda i: (0, i, 0))],
        out_specs=pl.BlockSpec((_B, D), lambda i: (i, 0)),
        out_shape=jax.ShapeDtypeStruct((N, D), jnp.float32),
    )(p)


@functools.partial(
    pl.kernel,
    out_type=jax.ShapeDtypeStruct((2, NA, TW), jnp.float32),
    mesh=plsc.VectorSubcoreMesh(core_axis_name="c", subcore_axis_name="s"),
    compiler_params=pltpu.CompilerParams(
        use_tc_tiling_on_sc=False, needs_layout_passes=False),
    scratch_types=[
        pltpu.VMEM((NCHX, CW), jnp.int16),
        pltpu.VMEM((NCHX, CW), jnp.int16),
        pltpu.VMEM((2, 2, CW), jnp.int32),
        pltpu.VMEM((CW, TW), jnp.float32),
        pltpu.VMEM((CW,), jnp.float32),
        pltpu.VMEM((2, HW), jnp.float32),
        pltpu.VMEM_SHARED((NA, TW), jnp.float32),
        pltpu.SemaphoreType.DMA,
        pltpu.SemaphoreType.DMA,
        pltpu.SemaphoreType.DMA,
        pltpu.SemaphoreType.DMA,
    ],
)
def _sc_aggregate(ztab, edt, src2d, dst2d, out, srcv16, dstv16, idxc, rows,
                  exv, edh, acc, semg0, semg1, sems0, sems1):
    c = lax.axis_index("c")
    s = lax.axis_index("s")
    wid = c * 16 + s
    nch = jnp.where(c == 0, NCH0, NCH1)
    iota16 = lax.iota(jnp.int32, 16)
    zero16 = jnp.zeros((16,), jnp.float32)
    semg = (semg0, semg1)
    sems = (sems0, sems1)

    def _zrow(i, carry):
        for r in range(TW // 16):
            rows[i, pl.ds(r * 16, 16)] = zero16
        return carry

    lax.fori_loop(0, CW, _zrow, 0)
    pieces = [(t * CW, CW) for t in range(RPS // CW)] + [
        (RPS // CW * CW, RPS % CW)]
    for rs, rn in pieces:
        pltpu.sync_copy(rows.at[pl.ds(0, rn)],
                        acc.at[pl.ds(s * RPS + rs, rn)])
    plsc.subcore_barrier()

    pltpu.sync_copy(src2d.at[wid], srcv16)
    pltpu.sync_copy(dst2d.at[wid], dstv16)

    def _convert(cc, p):
        for g in range(CW // 32):
            for src_sel, v16 in ((0, srcv16), (1, dstv16)):
                pk = v16[cc, pl.ds(32 * g, 32)]
                a, b = plsc.unpack(pk, format=plsc.PackFormat.INTERLEAVED,
                                   preferred_element_type=jnp.int32)
                idxc[p, src_sel, pl.ds(32 * g, 16)] = a
                idxc[p, src_sel, pl.ds(32 * g + 16, 16)] = b

    def _gather_half(p, b, buf):
        pltpu.async_copy(ztab.at[idxc.at[p, 0, pl.ds(b * HW, HW)]],
                         rows.at[pl.ds(buf * HW, HW)], semg[buf])
        pltpu.async_copy(edt.at[idxc.at[p, 1, pl.ds(b * HW, HW)]],
                         edh.at[buf], semg[buf])

    def _wait_gather(buf):
        pltpu.make_async_copy(ztab.at[idxc.at[0, 0, pl.ds(0, HW)]],
                              rows.at[pl.ds(buf * HW, HW)],
                              semg[buf]).wait()
        pltpu.make_async_copy(edt.at[idxc.at[0, 1, pl.ds(0, HW)]],
                              edh.at[buf], semg[buf]).wait()

    def _start_scatter(p, b, buf):
        pltpu.async_copy(rows.at[pl.ds(buf * HW, HW)],
                         acc.at[idxc.at[p, 1, pl.ds(b * HW, HW)]],
                         sems[buf], add=True)

    def _wait_scatter(buf):
        pltpu.make_async_copy(rows.at[pl.ds(buf * HW, HW)],
                              acc.at[idxc.at[0, 1, pl.ds(0, HW)]],
                              sems[buf]).wait()

    def _compute_half(b):
        for k in range(HW // 16):
            es16 = plsc.load_gather(
                rows, [iota16 + (b * HW + k * 16),
                       jnp.full((16,), D, jnp.int32)])
            d16 = edh[b, pl.ds(k * 16, 16)]
            raw = es16 + d16
            e16 = jnp.where(raw > 0, raw, 0.2 * raw)
            exv[pl.ds(b * HW + k * 16, 16)] = jnp.exp(e16)

        def _scale(ei, carry2):
            bc = plsc.load_gather(exv, [jnp.full((16,), 0, jnp.int32) + ei])
            for r in range(D // 16):
                rows[ei, pl.ds(r * 16, 16)] = (
                    rows[ei, pl.ds(r * 16, 16)] * bc)
            rows[ei, pl.ds(D, 16)] = bc
            return carry2

        lax.fori_loop(b * HW, b * HW + HW, _scale, 0)

    _convert(0, 0)
    _gather_half(0, 0, 0)

    def _pair(g, carry):
        for u in range(2):
            cc = 2 * g + u
            if u == 0:
                @pl.when(g > 0)
                def _():
                    _wait_scatter(1)
            else:
                _wait_scatter(1)
            _gather_half(u, 1, 1)
            _wait_gather(0)
            _compute_half(0)
            _start_scatter(u, 0, 0)
            _wait_scatter(0)

            @pl.when(cc < nch - 1)
            def _():
                _convert(cc + 1, 1 - u)
                _gather_half(1 - u, 0, 0)
            _wait_gather(1)
            _compute_half(1)
            _start_scatter(u, 1, 1)
        return carry

    lax.fori_loop(0, nch // 2, _pair, 0)
    _wait_scatter(1)
    plsc.subcore_barrier()

    for rs, rn in pieces:
        pltpu.sync_copy(acc.at[pl.ds(s * RPS + rs, rn)],
                        rows.at[pl.ds(0, rn)])
        pltpu.sync_copy(rows.at[pl.ds(0, rn)],
                        out.at[c, pl.ds(s * RPS + rs, rn)])


def _stage_indices(v, fill):
    s0 = v[:SPLIT].reshape(16, NCH0, CW)
    s1 = jnp.pad(v[SPLIT:], (0, 16 * NCH1 * CW - (E - SPLIT)),
                 constant_values=fill).reshape(16, NCH1, CW)
    s1 = jnp.pad(s1, ((0, 0), (0, NCHX - NCH1), (0, 0)),
                 constant_values=fill)
    return jnp.concatenate([s0, s1], axis=0).astype(jnp.int16)


def kernel(feature, edge_index, W1, a1, W2, a2, W3, a3, W4, a4):
    src2d = _stage_indices(edge_index[0], 0)
    dst2d = _stage_indices(edge_index[1], N)

    def a2col(a):
        return jnp.stack([a[:D], a[D:]], axis=1)

    def edpad(ed):
        return jnp.pad(ed[:, 0], (0, NA - N))

    zt, ed = _tc_proj(feature, W1, a2col(a1))
    p = _sc_aggregate(zt, edpad(ed), src2d, dst2d)
    for w, a in ((W2, a2), (W3, a3), (W4, a4)):
        zt, ed = _tc_comb_proj(p, w, a2col(a))
        p = _sc_aggregate(zt, edpad(ed), src2d, dst2d)
    return _tc_comb_last(p)


# speedup vs baseline: 30.5326x; 1.0516x over previous
import functools

import jax
import jax.numpy as jnp
from jax import lax
from jax.experimental import pallas as pl
from jax.experimental.pallas import tpu as pltpu
from jax.experimental.pallas import tpu_sc as plsc

N = 10000
E = 320000
D = 128
NT = N + 16
CW = 128
HW = CW // 2
NCH0 = 96
NCH1 = 62
NCHX = NCH0
SPLIT = 16 * NCH0 * CW
EPAD = 16 * (NCH0 + NCH1) * CW
NA = 10112
RPS = NA // 16


def _proj_body(h, w_ref, a_ref, zt_ref, e2_ref):
    z = jnp.dot(h, w_ref[...], preferred_element_type=jnp.float32)
    e2 = jnp.dot(z, a_ref[...], preferred_element_type=jnp.float32)
    zt_ref[...] = z
    e2_ref[...] = e2


def _tc_proj_kernel(h_ref, w_ref, a_ref, zt_ref, e2_ref):
    _proj_body(h_ref[...], w_ref, a_ref, zt_ref, e2_ref)


def _combine(p_ref, dn_ref):
    num = p_ref[0] + p_ref[1]
    den = dn_ref[0, :, 0] + dn_ref[1, :, 0]
    return num / (den + 1e-9)[:, None]


def _tc_comb_proj_kernel(p_ref, dn_ref, w_ref, a_ref, zt_ref, e2_ref):
    h = jnp.maximum(_combine(p_ref, dn_ref), 0.0)
    _proj_body(h, w_ref, a_ref, zt_ref, e2_ref)


def _tc_comb_last_kernel(p_ref, dn_ref, h_ref):
    h_ref[...] = _combine(p_ref, dn_ref)


_B = 1000


_ZOUT = [
    pl.BlockSpec((_B, D), lambda i: (i, 0)),
    pl.BlockSpec((_B, 2), lambda i: (i, 0)),
]
_ZSHP = [
    jax.ShapeDtypeStruct((N, D), jnp.float32),
    jax.ShapeDtypeStruct((N, 2), jnp.float32),
]


def _tc_proj(h, w, a2):
    return pl.pallas_call(
        _tc_proj_kernel,
        grid=(N // _B,),
        in_specs=[
            pl.BlockSpec((_B, D), lambda i: (i, 0)),
            pl.BlockSpec((D, D), lambda i: (0, 0)),
            pl.BlockSpec((D, 2), lambda i: (0, 0)),
        ],
        out_specs=_ZOUT,
        out_shape=_ZSHP,
    )(h, w, a2)


_PIN = [
    pl.BlockSpec((2, _B, D), lambda i: (0, i, 0)),
    pl.BlockSpec((2, _B, 1), lambda i: (0, i, 0)),
]


def _tc_comb_proj(p, dn, w, a2):
    return pl.pallas_call(
        _tc_comb_proj_kernel,
        grid=(N // _B,),
        in_specs=_PIN + [
            pl.BlockSpec((D, D), lambda i: (0, 0)),
            pl.BlockSpec((D, 2), lambda i: (0, 0)),
        ],
        out_specs=_ZOUT,
        out_shape=_ZSHP,
    )(p, dn, w, a2)


def _tc_comb_last(p, dn):
    return pl.pallas_call(
        _tc_comb_last_kernel,
        grid=(N // _B,),
        in_specs=_PIN,
        out_specs=pl.BlockSpec((_B, D), lambda i: (i, 0)),
        out_shape=jax.ShapeDtypeStruct((N, D), jnp.float32),
    )(p, dn)


@functools.partial(
    pl.kernel,
    out_type=(
        jax.ShapeDtypeStruct((2, NA, D), jnp.float32),
        jax.ShapeDtypeStruct((2, NA), jnp.float32),
    ),
    mesh=plsc.VectorSubcoreMesh(core_axis_name="c", subcore_axis_name="s"),
    compiler_params=pltpu.CompilerParams(
        use_tc_tiling_on_sc=False, needs_layout_passes=False),
    scratch_types=[
        pltpu.VMEM((NCHX, CW), jnp.int16),
        pltpu.VMEM((NCHX, CW), jnp.int16),
        pltpu.VMEM((2, 2, CW), jnp.int32),
        pltpu.VMEM((CW, D), jnp.float32),
        pltpu.VMEM((CW,), jnp.float32),
        pltpu.VMEM((NT,), jnp.float32),
        pltpu.VMEM((NT,), jnp.float32),
        pltpu.VMEM_SHARED((NA, D), jnp.float32),
        pltpu.VMEM_SHARED((NA,), jnp.float32),
        pltpu.SemaphoreType.DMA,
        pltpu.SemaphoreType.DMA,
        pltpu.SemaphoreType.DMA,
        pltpu.SemaphoreType.DMA,
    ],
)
def _sc_aggregate(ztab, est, edt, src2d, dst2d, out, outd, srcv16, dstv16,
                  idxc, rows, exv, estab, edtab, acc, dacc,
                  semg0, semg1, sems0, sems1):
    c = lax.axis_index("c")
    s = lax.axis_index("s")
    wid = c * 16 + s
    nch = jnp.where(c == 0, NCH0, NCH1)
    zero16 = jnp.zeros((16,), jnp.float32)
    semg = (semg0, semg1)
    sems = (sems0, sems1)

    def _zrow(i, carry):
        for r in range(D // 16):
            rows[i, pl.ds(r * 16, 16)] = zero16
        return carry

    lax.fori_loop(0, CW, _zrow, 0)
    pieces = [(t * CW, CW) for t in range(RPS // CW)] + [
        (RPS // CW * CW, RPS % CW)]
    for rs, rn in pieces:
        pltpu.sync_copy(rows.at[pl.ds(0, rn)],
                        acc.at[pl.ds(s * RPS + rs, rn)])
        pltpu.sync_copy(rows.at[0, pl.ds(0, rn)],
                        dacc.at[pl.ds(s * RPS + rs, rn)])
    plsc.subcore_barrier()

    pltpu.sync_copy(src2d.at[wid], srcv16)
    pltpu.sync_copy(dst2d.at[wid], dstv16)
    pltpu.sync_copy(est, estab)
    pltpu.sync_copy(edt, edtab)

    def _convert(cc, p):
        for g in range(CW // 32):
            for src_sel, v16 in ((0, srcv16), (1, dstv16)):
                pk = v16[cc, pl.ds(32 * g, 32)]
                a, b = plsc.unpack(pk, format=plsc.PackFormat.INTERLEAVED,
                                   preferred_element_type=jnp.int32)
                idxc[p, src_sel, pl.ds(32 * g, 16)] = a
                idxc[p, src_sel, pl.ds(32 * g + 16, 16)] = b

    def _gather_half(p, b, buf):
        pltpu.async_copy(ztab.at[idxc.at[p, 0, pl.ds(b * HW, HW)]],
                         rows.at[pl.ds(buf * HW, HW)], semg[buf])

    def _wait_gather(buf):
        pltpu.make_async_copy(ztab.at[idxc.at[0, 0, pl.ds(0, HW)]],
                              rows.at[pl.ds(buf * HW, HW)],
                              semg[buf]).wait()

    def _start_scatter(p, b, buf):
        pltpu.async_copy(rows.at[pl.ds(buf * HW, HW)],
                         acc.at[idxc.at[p, 1, pl.ds(b * HW, HW)]],
                         sems[buf], add=True)
        pltpu.async_copy(exv.at[pl.ds(buf * HW, HW)],
                         dacc.at[idxc.at[p, 1, pl.ds(b * HW, HW)]],
                         sems[buf], add=True)

    def _wait_scatter(buf):
        pltpu.make_async_copy(rows.at[pl.ds(buf * HW, HW)],
                              acc.at[idxc.at[0, 1, pl.ds(0, HW)]],
                              sems[buf]).wait()
        pltpu.make_async_copy(exv.at[pl.ds(buf * HW, HW)],
                              dacc.at[idxc.at[0, 1, pl.ds(0, HW)]],
                              sems[buf]).wait()

    def _compute_half(p, b):
        for k in range(HW // 16):
            s16 = idxc[p, 0, pl.ds(b * HW + k * 16, 16)]
            d16i = idxc[p, 1, pl.ds(b * HW + k * 16, 16)]
            raw = (plsc.load_gather(estab, [s16]) +
                   plsc.load_gather(edtab, [d16i]))
            e16 = jnp.where(raw > 0, raw, 0.2 * raw)
            exv[pl.ds(b * HW + k * 16, 16)] = jnp.exp(e16)

        def _scale(ei, carry2):
            bc = plsc.load_gather(exv, [jnp.full((16,), 0, jnp.int32) + ei])
            for r in range(D // 16):
                rows[ei, pl.ds(r * 16, 16)] = (
                    rows[ei, pl.ds(r * 16, 16)] * bc)
            return carry2

        lax.fori_loop(b * HW, b * HW + HW, _scale, 0)

    _convert(0, 0)
    _gather_half(0, 0, 0)

    def _pair(g, carry):
        for u in range(2):
            cc = 2 * g + u
            if u == 0:
                @pl.when(g > 0)
                def _():
                    _wait_scatter(1)
            else:
                _wait_scatter(1)
            _gather_half(u, 1, 1)
            _wait_gather(0)
            _compute_half(u, 0)
            _start_scatter(u, 0, 0)
            _wait_scatter(0)

            @pl.when(cc < nch - 1)
            def _():
                _convert(cc + 1, 1 - u)
                _gather_half(1 - u, 0, 0)
            _wait_gather(1)
            _compute_half(u, 1)
            _start_scatter(u, 1, 1)
        return carry

    lax.fori_loop(0, nch // 2, _pair, 0)
    _wait_scatter(1)
    plsc.subcore_barrier()

    for rs, rn in pieces:
        pltpu.sync_copy(acc.at[pl.ds(s * RPS + rs, rn)],
                        rows.at[pl.ds(0, rn)])
        pltpu.sync_copy(rows.at[pl.ds(0, rn)],
                        out.at[c, pl.ds(s * RPS + rs, rn)])
        pltpu.sync_copy(dacc.at[pl.ds(s * RPS + rs, rn)],
                        rows.at[0, pl.ds(0, rn)])
        pltpu.sync_copy(rows.at[0, pl.ds(0, rn)],
                        outd.at[c, pl.ds(s * RPS + rs, rn)])


def _stage_indices(v, fill):
    s0 = v[:SPLIT].reshape(16, NCH0, CW)
    s1 = jnp.pad(v[SPLIT:], (0, 16 * NCH1 * CW - (E - SPLIT)),
                 constant_values=fill).reshape(16, NCH1, CW)
    s1 = jnp.pad(s1, ((0, 0), (0, NCHX - NCH1), (0, 0)),
                 constant_values=fill)
    return jnp.concatenate([s0, s1], axis=0).astype(jnp.int16)


def kernel(feature, edge_index, W1, a1, W2, a2, W3, a3, W4, a4):
    src2d = _stage_indices(edge_index[0], 0)
    dst2d = _stage_indices(edge_index[1], N)

    def a2col(a):
        return jnp.stack([a[:D], a[D:]], axis=1)

    def tables(e2):
        return (jnp.pad(e2[:, 0], (0, NT - N)),
                jnp.pad(e2[:, 1], (0, NT - N)))

    zt, e2 = _tc_proj(feature, W1, a2col(a1))
    p, dn = _sc_aggregate(zt, *tables(e2), src2d, dst2d)
    for w, a in ((W2, a2), (W3, a3), (W4, a4)):
        zt, e2 = _tc_comb_proj(p, dn[..., None], w, a2col(a))
        p, dn = _sc_aggregate(zt, *tables(e2), src2d, dst2d)
    return _tc_comb_last(p, dn[..., None])
